# baseline (speedup 1.0000x reference)
import jax
import jax.numpy as jnp
from jax import lax
from jax.experimental import pallas as pl
from jax.experimental.pallas import tpu as pltpu
from jax.experimental.pallas import tpu_sc as plsc

VOCAB = 1000000
EMB_DIM = 64
PAD_DIM = 128
BATCH = 4096
HIST = 50

NC = 2
NS = 16
NW = NC * NS

TOKENS = BATCH * HIST
B_PER_W = TOKENS // NW
K = 128
NCHUNK = B_PER_W // K
NBUF = 5
F = 3
NGRP = NCHUNK // NBUF


def _emb_body(tok_hbm, w_hbm, out_hbm, idx_v,
              r0, r1, r2, r3, r4, g0, g1, g2, g3, g4,
              o0, o1, o2, o3, o4):
    rows = [r0, r1, r2, r3, r4]
    gsem = [g0, g1, g2, g3, g4]
    osem = [o0, o1, o2, o3, o4]
    c = lax.axis_index("c")
    s = lax.axis_index("s")
    wid = s * NC + c
    pltpu.sync_copy(tok_hbm.at[wid], idx_v)
    obase = wid * B_PER_W

    def fire_gather(j, b):
        pltpu.async_copy(w_hbm.at[idx_v.at[j]], rows[b], gsem[b])

    def wait_gather(b):
        pltpu.make_async_copy(w_hbm.at[pl.ds(0, K)], rows[b],
                              gsem[b]).wait()

    def fire_out(j, b):
        pltpu.async_copy(rows[b], out_hbm.at[pl.ds(obase + j * K, K)],
                         osem[b])

    def wait_out(b):
        pltpu.make_async_copy(out_hbm.at[pl.ds(0, K)], rows[b],
                              osem[b]).wait()

    for j in range(F):
        fire_gather(j, j)
    for j in range(NBUF):
        bq = j + F
        if bq < NBUF:
            fire_gather(bq, bq)
        else:
            wait_out(bq - NBUF)
            fire_gather(bq, bq - NBUF)
        wait_gather(j)
        fire_out(j, j)

    @pl.loop(1, NGRP - 1)
    def _(g):
        jg = g * NBUF
        for b in range(NBUF):
            bq = (b + F) % NBUF
            wait_out(bq)
            fire_gather(jg + b + F, bq)
            wait_gather(b)
            fire_out(jg + b, b)

    jg = (NGRP - 1) * NBUF
    for b in range(NBUF):
        if jg + b + F < NCHUNK:
            bq = (b + F) % NBUF
            wait_out(bq)
            fire_gather(jg + b + F, bq)
        wait_gather(b)
        fire_out(jg + b, b)
    for b in range(NBUF):
        wait_out(b)


@jax.jit
def _emb_lookup(tok3, w2):
    mesh = plsc.VectorSubcoreMesh(core_axis_name="c", subcore_axis_name="s")
    fn = pl.kernel(
        _emb_body,
        out_type=jax.ShapeDtypeStruct((TOKENS, PAD_DIM), jnp.float32),
        mesh=mesh,
        scratch_types=(
            [pltpu.VMEM((NCHUNK, K), jnp.int32)]
            + [pltpu.VMEM((K, PAD_DIM), jnp.float32) for _ in range(NBUF)]
            + [pltpu.SemaphoreType.DMA for _ in range(2 * NBUF)]
        ),
        compiler_params=pltpu.CompilerParams(use_tc_tiling_on_sc=False),
    )
    return fn(tok3, w2)


def kernel(token, weight):
    tok = token.reshape(-1).astype(jnp.int32)
    tok3 = (tok >> 1).reshape(NW, NCHUNK, K)
    w2 = weight.reshape(VOCAB // 2, PAD_DIM)
    outp = _emb_lookup(tok3, w2)
    odd = (tok & 1)[:, None].astype(jnp.bool_)
    out = jnp.where(odd, outp[:, EMB_DIM:], outp[:, :EMB_DIM])
    return out.reshape(BATCH, HIST, EMB_DIM)


# speedup vs baseline: 1.0010x; 1.0010x over previous
import jax
import jax.numpy as jnp
from jax import lax
from jax.experimental import pallas as pl
from jax.experimental.pallas import tpu as pltpu
from jax.experimental.pallas import tpu_sc as plsc

VOCAB = 1000000
EMB_DIM = 64
PAD_DIM = 128
BATCH = 4096
HIST = 50

NC = 2
NS = 16
NW = NC * NS

TOKENS = BATCH * HIST
B_PER_W = TOKENS // NW
K = 128
NCHUNK = B_PER_W // K
NBUF = 5
F = 3
NGRP = NCHUNK // NBUF


def _emb_body(tok_hbm, w_hbm, out_hbm, idx_v,
              r0, r1, r2, r3, r4, g0, g1, g2, g3, g4,
              o0, o1, o2, o3, o4):
    rows = [r0, r1, r2, r3, r4]
    gsem = [g0, g1, g2, g3, g4]
    osem = [o0, o1, o2, o3, o4]
    c = lax.axis_index("c")
    s = lax.axis_index("s")
    wid = s * NC + c
    pltpu.sync_copy(tok_hbm.at[wid], idx_v)
    obase = wid * B_PER_W

    def fire_gather(j, b):
        pltpu.async_copy(w_hbm.at[idx_v.at[j]], rows[b], gsem[b])

    def wait_gather(b):
        pltpu.make_async_copy(w_hbm.at[pl.ds(0, K)], rows[b],
                              gsem[b]).wait()

    def fire_out(j, b):
        pltpu.async_copy(rows[b], out_hbm.at[pl.ds(obase + j * K, K)],
                         osem[b])

    def wait_out(b):
        pltpu.make_async_copy(out_hbm.at[pl.ds(0, K)], rows[b],
                              osem[b]).wait()

    for j in range(F):
        fire_gather(j, j)
    for j in range(NBUF):
        bq = j + F
        if bq < NBUF:
            fire_gather(bq, bq)
        else:
            wait_out(bq - NBUF)
            fire_gather(bq, bq - NBUF)
        wait_gather(j)
        fire_out(j, j)

    @pl.loop(1, NGRP - 1)
    def _(g):
        jg = g * NBUF
        for b in range(NBUF):
            bq = (b + F) % NBUF
            wait_out(bq)
            fire_gather(jg + b + F, bq)
            wait_gather(b)
            fire_out(jg + b, b)

    jg = (NGRP - 1) * NBUF
    for b in range(NBUF):
        if jg + b + F < NCHUNK:
            bq = (b + F) % NBUF
            wait_out(bq)
            fire_gather(jg + b + F, bq)
        wait_gather(b)
        fire_out(jg + b, b)
    for b in range(NBUF):
        wait_out(b)


@jax.jit
def _emb_lookup(tok3, w2):
    mesh = plsc.VectorSubcoreMesh(core_axis_name="c", subcore_axis_name="s")
    fn = pl.kernel(
        _emb_body,
        out_type=jax.ShapeDtypeStruct((TOKENS, PAD_DIM), jnp.float32),
        mesh=mesh,
        scratch_types=(
            [pltpu.VMEM((NCHUNK, K), jnp.int32)]
            + [pltpu.VMEM((K, PAD_DIM), jnp.float32) for _ in range(NBUF)]
            + [pltpu.SemaphoreType.DMA for _ in range(2 * NBUF)]
        ),
        compiler_params=pltpu.CompilerParams(use_tc_tiling_on_sc=True),
    )
    return fn(tok3, w2)


def kernel(token, weight):
    tok = token.reshape(-1).astype(jnp.int32)
    tok3 = (tok >> 1).reshape(NW, NCHUNK, K)
    w2 = weight.reshape(VOCAB // 2, PAD_DIM)
    outp = _emb_lookup(tok3, w2)
    odd = (tok & 1)[:, None].astype(jnp.bool_)
    out = jnp.where(odd, outp[:, EMB_DIM:], outp[:, :EMB_DIM])
    return out.reshape(BATCH, HIST, EMB_DIM)


# speedup vs baseline: 1.2594x; 1.2581x over previous
import jax
import jax.numpy as jnp
from jax import lax
from jax.experimental import pallas as pl
from jax.experimental.pallas import tpu as pltpu
from jax.experimental.pallas import tpu_sc as plsc

VOCAB = 1000000
EMB_DIM = 64
PAD_DIM = 128
BATCH = 4096
HIST = 50

NC = 2
NS = 16
NW = NC * NS

TOKENS = BATCH * HIST
B_PER_W = TOKENS // NW
K = 128
NCHUNK = B_PER_W // K
NBUF = 5
F = 3
NGRP = NCHUNK // NBUF


def _emb_body(tok_hbm, w_hbm, out_hbm, idx_v,
              r0, r1, r2, r3, r4, g0, g1, g2, g3, g4,
              o0, o1, o2, o3, o4):
    rows = [r0, r1, r2, r3, r4]
    gsem = [g0, g1, g2, g3, g4]
    osem = [o0, o1, o2, o3, o4]
    c = lax.axis_index("c")
    s = lax.axis_index("s")
    wid = s * NC + c
    pltpu.sync_copy(tok_hbm.at[wid], idx_v)
    base = wid * B_PER_W

    def fire_gather(j, b):
        pltpu.async_copy(w_hbm.at[idx_v.at[j]], rows[b], gsem[b])

    def wait_gather(b):
        pltpu.make_async_copy(w_hbm.at[pl.ds(0, K)], rows[b], gsem[b]).wait()

    def fire_out(j, b):
        pltpu.async_copy(rows[b], out_hbm.at[pl.ds(base + j * K, K)],
                         osem[b])

    def wait_out(b):
        pltpu.make_async_copy(out_hbm.at[pl.ds(base, K)], rows[b],
                              osem[b]).wait()

    for j in range(F):
        fire_gather(j, j)
    for j in range(NBUF):
        bq = j + F
        if bq < NBUF:
            fire_gather(bq, bq)
        else:
            wait_out(bq - NBUF)
            fire_gather(bq, bq - NBUF)
        wait_gather(j)
        fire_out(j, j)

    @pl.loop(1, NGRP - 1)
    def _(g):
        jg = g * NBUF
        for b in range(NBUF):
            bq = (b + F) % NBUF
            wait_out(bq)
            fire_gather(jg + b + F, bq)
            wait_gather(b)
            fire_out(jg + b, b)

    jg = (NGRP - 1) * NBUF
    for b in range(NBUF):
        if jg + b + F < NCHUNK:
            bq = (b + F) % NBUF
            wait_out(bq)
            fire_gather(jg + b + F, bq)
        wait_gather(b)
        fire_out(jg + b, b)
    for b in range(NBUF):
        wait_out(b)


@jax.jit
def _emb_lookup(tok3, w128):
    mesh = plsc.VectorSubcoreMesh(core_axis_name="c", subcore_axis_name="s")
    fn = pl.kernel(
        _emb_body,
        out_type=jax.ShapeDtypeStruct((TOKENS, PAD_DIM), jnp.float32),
        mesh=mesh,
        scratch_types=(
            [pltpu.VMEM((NCHUNK, K), jnp.int32)]
            + [pltpu.VMEM((K, PAD_DIM), jnp.float32) for _ in range(NBUF)]
            + [pltpu.SemaphoreType.DMA for _ in range(2 * NBUF)]
        ),
        compiler_params=pltpu.CompilerParams(use_tc_tiling_on_sc=True),
    )
    return fn(tok3, w128)


def kernel(token, weight):
    tok3 = token.reshape(NW, NCHUNK, K).astype(jnp.int32)
    w128 = jnp.pad(weight, ((0, 0), (0, PAD_DIM - EMB_DIM)))
    out = _emb_lookup(tok3, w128)
    return out[:, :EMB_DIM].reshape(BATCH, HIST, EMB_DIM)
